# baseline (speedup 1.0000x reference)
import functools

import jax
import jax.numpy as jnp
from jax import lax
from jax.experimental import pallas as pl
from jax.experimental.pallas import tpu as pltpu
from jax.experimental.pallas import tpu_sc as plsc

N_NODES = 50000
E_EDGES = 800000
H_DIM = 64
ED_DIM = 16
N_LAYERS = 6
QC = 16
NQ = 4

_NC = 2
_NS = 16
_ET = E_EDGES // _NS
_K = 80
_NCHUNK = _ET // _K
_ZR = 3128
_ZR_LAST = N_NODES - (_NS - 1) * _ZR

_TE = 4000
_TN = 2000


def _silu(x):
    return x / (1.0 + jnp.exp(-x))


_NB = 5


def _sc_edge_body(l, hm_hbm, efp_hbm, sd_hbm, agg_hbm, *scr):
    idx2 = scr[0:_NB]
    gbuf = scr[_NB:2 * _NB]
    ebuf = scr[2 * _NB:3 * _NB]
    zbuf = scr[3 * _NB]
    sem_ld = scr[3 * _NB + 1:4 * _NB + 1]
    sem_g = scr[4 * _NB + 1:5 * _NB + 1]
    sem_sc = scr[5 * _NB + 1:6 * _NB + 1]
    aggsh = scr[6 * _NB + 1]

    c = lax.axis_index("c")
    s = lax.axis_index("s")
    base = s * _ET

    def zrow(j, carry):
        for r in range(4):
            zbuf[j * 4 + r, 0:16] = jnp.zeros((16,), jnp.float32)
        return carry

    lax.fori_loop(0, _ZR // 4, zrow, 0)

    for p in range(2):
        q = 2 * p + c
        col = l * H_DIM + p * 2 * QC

        def start_loads(j, b):
            jg = s * _NCHUNK + j
            pltpu.async_copy(sd_hbm.at[q, jg], idx2[b], sem_ld[b])
            pltpu.async_copy(
                efp_hbm.at[pl.ds(base + j * _K, _K), pl.ds(col + c * QC, QC)],
                ebuf[b], sem_ld[b])

        def wait_loads(j, b):
            jg = s * _NCHUNK + j
            pltpu.make_async_copy(sd_hbm.at[q, jg], idx2[b], sem_ld[b]).wait()
            pltpu.make_async_copy(
                efp_hbm.at[pl.ds(base + j * _K, _K), pl.ds(col + c * QC, QC)],
                ebuf[b], sem_ld[b]).wait()

        def start_gather(b):
            pltpu.async_copy(hm_hbm.at[idx2[b].at[0]], gbuf[b], sem_g[b])

        def wait_gather(b):
            pltpu.make_async_copy(hm_hbm.at[idx2[b].at[0]], gbuf[b],
                                  sem_g[b]).wait()

        def start_scatter(b):
            pltpu.sync_copy(gbuf[b], aggsh.at[idx2[b].at[1]], add=True)

        def wait_scatter(b):
            del b

        def compute(b):
            def frow(r, cc):
                for t in range(4):
                    x = gbuf[b][r * 4 + t, 0:16] + ebuf[b][r * 4 + t, 0:16]
                    gbuf[b][r * 4 + t, 0:16] = x / (1.0 + jnp.exp(-x))
                return cc

            lax.fori_loop(0, _K // 4, frow, 0)

        @pl.when(s < _NS - 1)
        def _():
            pltpu.sync_copy(zbuf, aggsh.at[pl.ds(s * _ZR, _ZR)])

        @pl.when(s == _NS - 1)
        def _():
            pltpu.sync_copy(zbuf.at[pl.ds(0, _ZR_LAST)],
                            aggsh.at[pl.ds((_NS - 1) * _ZR, _ZR_LAST)])

        plsc.subcore_barrier()

        start_loads(0, 0)
        start_loads(1, 1)
        wait_loads(0, 0)
        start_gather(0)

        def step(j, u):
            v1 = (u + 1) % _NB
            v2 = (u + 2) % _NB
            wait_gather(u)

            @pl.when(j + 1 < _NCHUNK)
            def _():
                wait_loads(j + 1, v1)
                start_gather(v1)

            compute(u)

            @pl.when(j >= 1)
            def _():
                wait_scatter((u + _NB - 1) % _NB)

            start_scatter(u)

            @pl.when(j + 2 < _NCHUNK)
            def _():
                start_loads(j + 2, v2)

        def group(g, carry):
            for u in range(_NB):
                step(g * _NB + u, u)
            return carry

        lax.fori_loop(0, _NCHUNK // _NB, group, 0)
        wait_scatter((_NCHUNK - 1) % _NB)

        plsc.subcore_barrier()

        @pl.when(s == 0)
        def _():
            pltpu.sync_copy(aggsh, agg_hbm.at[q])

        plsc.subcore_barrier()


def _make_sc_edge(l):
    return pl.kernel(
        functools.partial(_sc_edge_body, l),
        out_type=jax.ShapeDtypeStruct((NQ, N_NODES, QC), jnp.float32),
        mesh=plsc.VectorSubcoreMesh(core_axis_name="c", subcore_axis_name="s"),
        compiler_params=pltpu.CompilerParams(use_tc_tiling_on_sc=False),
        scratch_types=(
            [pltpu.VMEM((2, _K), jnp.int32) for _ in range(_NB)]
            + [pltpu.VMEM((_K, QC), jnp.float32) for _ in range(2 * _NB)]
            + [pltpu.VMEM((_ZR, QC), jnp.float32)]
            + [pltpu.SemaphoreType.DMA for _ in range(3 * _NB)]
            + [pltpu.VMEM_SHARED((N_NODES, QC), jnp.float32)]
        ),
    )


_SC_EDGE = [_make_sc_edge(l) for l in range(N_LAYERS)]


def _efp_body(ef_ref, w_ref, out_ref):
    out_ref[...] = jnp.dot(ef_ref[...], w_ref[...],
                           preferred_element_type=jnp.float32)


_efp_call = pl.pallas_call(
    _efp_body,
    grid=(E_EDGES // _TE,),
    in_specs=[pl.BlockSpec((_TE, ED_DIM), lambda i: (i, 0)),
              pl.BlockSpec((ED_DIM, N_LAYERS * H_DIM), lambda i: (0, 0))],
    out_specs=pl.BlockSpec((_TE, N_LAYERS * H_DIM), lambda i: (i, 0)),
    out_shape=jax.ShapeDtypeStruct((E_EDGES, N_LAYERS * H_DIM), jnp.float32),
)


def _init_body(win_ref, bin_ref, wma_ref, bm0_ref, h_ref, hm_ref):
    row = win_ref[...] + bin_ref[...]
    h_ref[...] = jnp.broadcast_to(row, (_TN, H_DIM))
    hmrow = jnp.dot(row, wma_ref[...], preferred_element_type=jnp.float32) + bm0_ref[...]
    for q in range(NQ):
        hm_ref[q] = jnp.broadcast_to(hmrow[:, q * QC:(q + 1) * QC], (_TN, QC))


_init_call = pl.pallas_call(
    _init_body,
    grid=(N_NODES // _TN,),
    in_specs=[pl.BlockSpec((1, H_DIM), lambda i: (0, 0)),
              pl.BlockSpec((1, H_DIM), lambda i: (0, 0)),
              pl.BlockSpec((H_DIM, H_DIM), lambda i: (0, 0)),
              pl.BlockSpec((1, H_DIM), lambda i: (0, 0))],
    out_specs=[pl.BlockSpec((_TN, H_DIM), lambda i: (i, 0)),
               pl.BlockSpec((NQ, _TN, QC), lambda i: (0, i, 0))],
    out_shape=[jax.ShapeDtypeStruct((N_NODES, H_DIM), jnp.float32),
               jax.ShapeDtypeStruct((NQ, N_NODES, QC), jnp.float32)],
)


def _update(h_ref, agg_ref, wu_ref, bu_ref):
    h = h_ref[...]
    u = jnp.dot(h, wu_ref[0:H_DIM], preferred_element_type=jnp.float32) + bu_ref[...]
    for q in range(NQ):
        u = u + jnp.dot(agg_ref[q], wu_ref[H_DIM + q * QC:H_DIM + (q + 1) * QC],
                        preferred_element_type=jnp.float32)
    return h + _silu(u)


def _mid_body(h_ref, agg_ref, wu_ref, bu_ref, wman_ref, bmn_ref, ho_ref, hmo_ref):
    hn = _update(h_ref, agg_ref, wu_ref, bu_ref)
    ho_ref[...] = hn
    hm = jnp.dot(hn, wman_ref[...], preferred_element_type=jnp.float32) + bmn_ref[...]
    for q in range(NQ):
        hmo_ref[q] = hm[:, q * QC:(q + 1) * QC]


_mid_call = pl.pallas_call(
    _mid_body,
    grid=(N_NODES // _TN,),
    in_specs=[pl.BlockSpec((_TN, H_DIM), lambda i: (i, 0)),
              pl.BlockSpec((NQ, _TN, QC), lambda i: (0, i, 0)),
              pl.BlockSpec((2 * H_DIM, H_DIM), lambda i: (0, 0)),
              pl.BlockSpec((1, H_DIM), lambda i: (0, 0)),
              pl.BlockSpec((H_DIM, H_DIM), lambda i: (0, 0)),
              pl.BlockSpec((1, H_DIM), lambda i: (0, 0))],
    out_specs=[pl.BlockSpec((_TN, H_DIM), lambda i: (i, 0)),
               pl.BlockSpec((NQ, _TN, QC), lambda i: (0, i, 0))],
    out_shape=[jax.ShapeDtypeStruct((N_NODES, H_DIM), jnp.float32),
               jax.ShapeDtypeStruct((NQ, N_NODES, QC), jnp.float32)],
)


def _final_body(h_ref, agg_ref, wu_ref, bu_ref, wr1_ref, br1_ref, wr2_ref,
                br2_ref, out_ref):
    hn = _update(h_ref, agg_ref, wu_ref, bu_ref)
    t = _silu(jnp.dot(hn, wr1_ref[...], preferred_element_type=jnp.float32)
              + br1_ref[...])
    out_ref[...] = jnp.dot(t, wr2_ref[...], preferred_element_type=jnp.float32) + br2_ref[...]


_final_call = pl.pallas_call(
    _final_body,
    grid=(N_NODES // _TN,),
    in_specs=[pl.BlockSpec((_TN, H_DIM), lambda i: (i, 0)),
              pl.BlockSpec((NQ, _TN, QC), lambda i: (0, i, 0)),
              pl.BlockSpec((2 * H_DIM, H_DIM), lambda i: (0, 0)),
              pl.BlockSpec((1, H_DIM), lambda i: (0, 0)),
              pl.BlockSpec((H_DIM, H_DIM), lambda i: (0, 0)),
              pl.BlockSpec((1, H_DIM), lambda i: (0, 0)),
              pl.BlockSpec((H_DIM, 1), lambda i: (0, 0)),
              pl.BlockSpec((1, 1), lambda i: (0, 0))],
    out_specs=pl.BlockSpec((_TN, 1), lambda i: (i, 0)),
    out_shape=jax.ShapeDtypeStruct((N_NODES, 1), jnp.float32),
)


def _softmax_body(x_ref, o_ref):
    x = x_ref[...]
    m = jnp.max(x)
    e = jnp.exp(x - m)
    o_ref[...] = e / jnp.sum(e)


_softmax_call = pl.pallas_call(
    _softmax_body,
    out_shape=jax.ShapeDtypeStruct((400, 125), jnp.float32),
)


def kernel(edge_index, edge_feat, N, W_in, b_in, Wm, bm, Wu, bu, Wr1, br1,
           Wr2, br2):
    del N
    src = edge_index[0].astype(jnp.int32)
    dst = edge_index[1].astype(jnp.int32)
    dst_c = jnp.broadcast_to(dst.reshape(1, E_EDGES // _K, _K),
                             (NQ, E_EDGES // _K, _K))
    src_c = jnp.stack([src + q * N_NODES for q in range(NQ)]
                      ).reshape(NQ, E_EDGES // _K, _K)
    sd = jnp.stack([src_c, dst_c], axis=2)

    wmb = jnp.concatenate([Wm[l, H_DIM:, :] for l in range(N_LAYERS)], axis=1)
    efp = _efp_call(edge_feat, wmb)

    h, hm = _init_call(W_in, b_in.reshape(1, H_DIM), Wm[0, :H_DIM, :], bm[0:1])

    logits = None
    for l in range(N_LAYERS):
        agg = _SC_EDGE[l](hm.reshape(NQ * N_NODES, QC), efp, sd)
        if l + 1 < N_LAYERS:
            h, hm = _mid_call(h, agg, Wu[l], bu[l:l + 1],
                              Wm[l + 1, :H_DIM, :], bm[l + 1:l + 2])
        else:
            logits = _final_call(h, agg, Wu[l], bu[l:l + 1], Wr1,
                                 br1.reshape(1, H_DIM), Wr2, br2.reshape(1, 1))

    p = _softmax_call(logits.reshape(400, 125))
    return p.reshape(N_NODES)


# speedup vs baseline: 1.1866x; 1.1866x over previous
import functools

import jax
import jax.numpy as jnp
from jax import lax
from jax.experimental import pallas as pl
from jax.experimental.pallas import tpu as pltpu
from jax.experimental.pallas import tpu_sc as plsc

N_NODES = 50000
E_EDGES = 800000
H_DIM = 64
ED_DIM = 16
N_LAYERS = 6
QC = 16
NQ = 4

_NC = 2
_NS = 16
_K = 128
_NCG = E_EDGES // _K
_NCHUNK = _NCG // _NS
_NEXTRA = _NCG - _NCHUNK * _NS
_ZR = 3128
_ZR_LAST = N_NODES - (_NS - 1) * _ZR

_TE = 4000
_TN = 2000


def _silu(x):
    return x / (1.0 + jnp.exp(-x))


_NB = 5


def _sc_edge_body(l, hm_hbm, efp_hbm, src4_hbm, dst_hbm, agg_hbm, *scr):
    idx_s = scr[0:_NB]
    idx_d = scr[_NB:2 * _NB]
    gbuf = scr[2 * _NB:3 * _NB]
    ebuf = scr[3 * _NB:4 * _NB]
    zbuf = scr[4 * _NB]
    sem_ld = scr[4 * _NB + 1:5 * _NB + 1]
    sem_g = scr[5 * _NB + 1:6 * _NB + 1]
    sem_sc = scr[6 * _NB + 1:7 * _NB + 1]
    aggsh = scr[7 * _NB + 1]

    c = lax.axis_index("c")
    s = lax.axis_index("s")
    base = s * _NCHUNK * _K

    def zrow(j, carry):
        for r in range(4):
            zbuf[j * 4 + r, 0:16] = jnp.zeros((16,), jnp.float32)
        return carry

    lax.fori_loop(0, _ZR // 4, zrow, 0)

    for p in range(2):
        q = 2 * p + c
        col = l * H_DIM + p * 2 * QC

        def start_loads(e0, b):
            pltpu.async_copy(src4_hbm.at[pl.ds(q * E_EDGES + e0, _K)],
                             idx_s[b], sem_ld[b])
            pltpu.async_copy(dst_hbm.at[pl.ds(e0, _K)], idx_d[b], sem_ld[b])
            pltpu.async_copy(
                efp_hbm.at[pl.ds(e0, _K), pl.ds(col + c * QC, QC)],
                ebuf[b], sem_ld[b])

        def wait_loads(e0, b):
            pltpu.make_async_copy(src4_hbm.at[pl.ds(q * E_EDGES + e0, _K)],
                                  idx_s[b], sem_ld[b]).wait()
            pltpu.make_async_copy(dst_hbm.at[pl.ds(e0, _K)], idx_d[b],
                                  sem_ld[b]).wait()
            pltpu.make_async_copy(
                efp_hbm.at[pl.ds(e0, _K), pl.ds(col + c * QC, QC)],
                ebuf[b], sem_ld[b]).wait()

        def start_gather(b):
            pltpu.async_copy(hm_hbm.at[idx_s[b]], gbuf[b], sem_g[b])

        def wait_gather(b):
            pltpu.make_async_copy(hm_hbm.at[idx_s[b]], gbuf[b], sem_g[b]).wait()

        def start_scatter(b):
            pltpu.sync_copy(gbuf[b], aggsh.at[idx_d[b]], add=True)

        def wait_scatter(b):
            del b

        def compute(b):
            def frow(r, cc):
                for t in range(4):
                    x = gbuf[b][r * 4 + t, 0:16] + ebuf[b][r * 4 + t, 0:16]
                    gbuf[b][r * 4 + t, 0:16] = x / (1.0 + jnp.exp(-x))
                return cc

            lax.fori_loop(0, _K // 4, frow, 0)

        @pl.when(s < _NS - 1)
        def _():
            pltpu.sync_copy(zbuf, aggsh.at[pl.ds(s * _ZR, _ZR)])

        @pl.when(s == _NS - 1)
        def _():
            pltpu.sync_copy(zbuf.at[pl.ds(0, _ZR_LAST)],
                            aggsh.at[pl.ds((_NS - 1) * _ZR, _ZR_LAST)])

        plsc.subcore_barrier()

        start_loads(base, 0)
        start_loads(base + _K, 1)
        wait_loads(base, 0)
        start_gather(0)

        def step(j, u):
            v1 = (u + 1) % _NB
            v2 = (u + 2) % _NB
            wait_gather(u)

            @pl.when(j + 1 < _NCHUNK)
            def _():
                wait_loads(base + (j + 1) * _K, v1)
                start_gather(v1)

            compute(u)
            start_scatter(u)

            @pl.when(j + 2 < _NCHUNK)
            def _():
                start_loads(base + (j + 2) * _K, v2)

        def group(g, carry):
            for u in range(_NB):
                step(g * _NB + u, u)
            return carry

        lax.fori_loop(0, _NCHUNK // _NB, group, 0)

        @pl.when(s < _NEXTRA)
        def _():
            ex0 = (_NS * _NCHUNK + s) * _K
            start_loads(ex0, 0)
            wait_loads(ex0, 0)
            start_gather(0)
            wait_gather(0)
            compute(0)
            start_scatter(0)

        plsc.subcore_barrier()

        @pl.when(s == 0)
        def _():
            pltpu.sync_copy(aggsh, agg_hbm.at[q])

        plsc.subcore_barrier()


def _make_sc_edge(l):
    return pl.kernel(
        functools.partial(_sc_edge_body, l),
        out_type=jax.ShapeDtypeStruct((NQ, N_NODES, QC), jnp.float32),
        mesh=plsc.VectorSubcoreMesh(core_axis_name="c", subcore_axis_name="s"),
        compiler_params=pltpu.CompilerParams(use_tc_tiling_on_sc=False),
        scratch_types=(
            [pltpu.VMEM((_K,), jnp.int32) for _ in range(2 * _NB)]
            + [pltpu.VMEM((_K, QC), jnp.float32) for _ in range(2 * _NB)]
            + [pltpu.VMEM((_ZR, QC), jnp.float32)]
            + [pltpu.SemaphoreType.DMA for _ in range(3 * _NB)]
            + [pltpu.VMEM_SHARED((N_NODES, QC), jnp.float32)]
        ),
    )


_SC_EDGE = [_make_sc_edge(l) for l in range(N_LAYERS)]


def _efp_body(ef_ref, w_ref, out_ref):
    out_ref[...] = jnp.dot(ef_ref[...], w_ref[...],
                           preferred_element_type=jnp.float32)


_efp_call = pl.pallas_call(
    _efp_body,
    grid=(E_EDGES // _TE,),
    in_specs=[pl.BlockSpec((_TE, ED_DIM), lambda i: (i, 0)),
              pl.BlockSpec((ED_DIM, N_LAYERS * H_DIM), lambda i: (0, 0))],
    out_specs=pl.BlockSpec((_TE, N_LAYERS * H_DIM), lambda i: (i, 0)),
    out_shape=jax.ShapeDtypeStruct((E_EDGES, N_LAYERS * H_DIM), jnp.float32),
)


def _init_body(win_ref, bin_ref, wma_ref, bm0_ref, h_ref, hm_ref):
    row = win_ref[...] + bin_ref[...]
    h_ref[...] = jnp.broadcast_to(row, (_TN, H_DIM))
    hmrow = jnp.dot(row, wma_ref[...], preferred_element_type=jnp.float32) + bm0_ref[...]
    for q in range(NQ):
        hm_ref[q] = jnp.broadcast_to(hmrow[:, q * QC:(q + 1) * QC], (_TN, QC))


_init_call = pl.pallas_call(
    _init_body,
    grid=(N_NODES // _TN,),
    in_specs=[pl.BlockSpec((1, H_DIM), lambda i: (0, 0)),
              pl.BlockSpec((1, H_DIM), lambda i: (0, 0)),
              pl.BlockSpec((H_DIM, H_DIM), lambda i: (0, 0)),
              pl.BlockSpec((1, H_DIM), lambda i: (0, 0))],
    out_specs=[pl.BlockSpec((_TN, H_DIM), lambda i: (i, 0)),
               pl.BlockSpec((NQ, _TN, QC), lambda i: (0, i, 0))],
    out_shape=[jax.ShapeDtypeStruct((N_NODES, H_DIM), jnp.float32),
               jax.ShapeDtypeStruct((NQ, N_NODES, QC), jnp.float32)],
)


def _update(h_ref, agg_ref, wu_ref, bu_ref):
    h = h_ref[...]
    u = jnp.dot(h, wu_ref[0:H_DIM], preferred_element_type=jnp.float32) + bu_ref[...]
    for q in range(NQ):
        u = u + jnp.dot(agg_ref[q], wu_ref[H_DIM + q * QC:H_DIM + (q + 1) * QC],
                        preferred_element_type=jnp.float32)
    return h + _silu(u)


def _mid_body(h_ref, agg_ref, wu_ref, bu_ref, wman_ref, bmn_ref, ho_ref, hmo_ref):
    hn = _update(h_ref, agg_ref, wu_ref, bu_ref)
    ho_ref[...] = hn
    hm = jnp.dot(hn, wman_ref[...], preferred_element_type=jnp.float32) + bmn_ref[...]
    for q in range(NQ):
        hmo_ref[q] = hm[:, q * QC:(q + 1) * QC]


_mid_call = pl.pallas_call(
    _mid_body,
    grid=(N_NODES // _TN,),
    in_specs=[pl.BlockSpec((_TN, H_DIM), lambda i: (i, 0)),
              pl.BlockSpec((NQ, _TN, QC), lambda i: (0, i, 0)),
              pl.BlockSpec((2 * H_DIM, H_DIM), lambda i: (0, 0)),
              pl.BlockSpec((1, H_DIM), lambda i: (0, 0)),
              pl.BlockSpec((H_DIM, H_DIM), lambda i: (0, 0)),
              pl.BlockSpec((1, H_DIM), lambda i: (0, 0))],
    out_specs=[pl.BlockSpec((_TN, H_DIM), lambda i: (i, 0)),
               pl.BlockSpec((NQ, _TN, QC), lambda i: (0, i, 0))],
    out_shape=[jax.ShapeDtypeStruct((N_NODES, H_DIM), jnp.float32),
               jax.ShapeDtypeStruct((NQ, N_NODES, QC), jnp.float32)],
)


def _final_body(h_ref, agg_ref, wu_ref, bu_ref, wr1_ref, br1_ref, wr2_ref,
                br2_ref, out_ref):
    hn = _update(h_ref, agg_ref, wu_ref, bu_ref)
    t = _silu(jnp.dot(hn, wr1_ref[...], preferred_element_type=jnp.float32)
              + br1_ref[...])
    out_ref[...] = jnp.dot(t, wr2_ref[...], preferred_element_type=jnp.float32) + br2_ref[...]


_final_call = pl.pallas_call(
    _final_body,
    grid=(N_NODES // _TN,),
    in_specs=[pl.BlockSpec((_TN, H_DIM), lambda i: (i, 0)),
              pl.BlockSpec((NQ, _TN, QC), lambda i: (0, i, 0)),
              pl.BlockSpec((2 * H_DIM, H_DIM), lambda i: (0, 0)),
              pl.BlockSpec((1, H_DIM), lambda i: (0, 0)),
              pl.BlockSpec((H_DIM, H_DIM), lambda i: (0, 0)),
              pl.BlockSpec((1, H_DIM), lambda i: (0, 0)),
              pl.BlockSpec((H_DIM, 1), lambda i: (0, 0)),
              pl.BlockSpec((1, 1), lambda i: (0, 0))],
    out_specs=pl.BlockSpec((_TN, 1), lambda i: (i, 0)),
    out_shape=jax.ShapeDtypeStruct((N_NODES, 1), jnp.float32),
)


def _softmax_body(x_ref, o_ref):
    x = x_ref[...]
    m = jnp.max(x)
    e = jnp.exp(x - m)
    o_ref[...] = e / jnp.sum(e)


_softmax_call = pl.pallas_call(
    _softmax_body,
    out_shape=jax.ShapeDtypeStruct((400, 125), jnp.float32),
)


def kernel(edge_index, edge_feat, N, W_in, b_in, Wm, bm, Wu, bu, Wr1, br1,
           Wr2, br2):
    del N
    src = edge_index[0].astype(jnp.int32)
    dst = edge_index[1].astype(jnp.int32)
    src4 = jnp.concatenate([src + q * N_NODES for q in range(NQ)])

    wmb = jnp.concatenate([Wm[l, H_DIM:, :] for l in range(N_LAYERS)], axis=1)
    efp = _efp_call(edge_feat, wmb)

    h, hm = _init_call(W_in, b_in.reshape(1, H_DIM), Wm[0, :H_DIM, :], bm[0:1])

    logits = None
    for l in range(N_LAYERS):
        agg = _SC_EDGE[l](hm.reshape(NQ * N_NODES, QC), efp, src4, dst)
        if l + 1 < N_LAYERS:
            h, hm = _mid_call(h, agg, Wu[l], bu[l:l + 1],
                              Wm[l + 1, :H_DIM, :], bm[l + 1:l + 2])
        else:
            logits = _final_call(h, agg, Wu[l], bu[l:l + 1], Wr1,
                                 br1.reshape(1, H_DIM), Wr2, br2.reshape(1, 1))

    p = _softmax_call(logits.reshape(400, 125))
    return p.reshape(N_NODES)


# speedup vs baseline: 1.3884x; 1.1700x over previous
import functools

import jax
import jax.numpy as jnp
from jax import lax
from jax.experimental import pallas as pl
from jax.experimental.pallas import tpu as pltpu
from jax.experimental.pallas import tpu_sc as plsc

N_NODES = 50000
E_EDGES = 800000
H_DIM = 64
ED_DIM = 16
N_LAYERS = 6
QC = 16
NQ = 4

_NC = 2
_NS = 16
_K = 128
_NCG = E_EDGES // _K
_NCHUNK = _NCG // _NS
_NEXTRA = _NCG - _NCHUNK * _NS
_ZR = 3128
_ZR_LAST = N_NODES - (_NS - 1) * _ZR

_TE = 4000
_TN = 2000


def _silu(x):
    return x / (1.0 + jnp.exp(-x))


_NB = 5


def _sc_edge_body(l, hm_hbm, efp_hbm, src4_hbm, dst_hbm, agg_hbm, *scr):
    first = l == 0
    idx_s = scr[0:_NB]
    idx_d = scr[_NB:2 * _NB]
    gbuf = scr[2 * _NB:3 * _NB]
    ebuf = scr[3 * _NB:4 * _NB]
    zbuf = scr[4 * _NB]
    sem_ld = scr[4 * _NB + 1:5 * _NB + 1]
    sem_g = scr[5 * _NB + 1:6 * _NB + 1]
    sem_sc = scr[6 * _NB + 1:7 * _NB + 1]
    aggsh = scr[7 * _NB + 1]
    hrow = scr[7 * _NB + 2]

    c = lax.axis_index("c")
    s = lax.axis_index("s")
    base = s * _NCHUNK * _K

    def zrow(j, carry):
        for r in range(4):
            zbuf[j * 4 + r, 0:16] = jnp.zeros((16,), jnp.float32)
        return carry

    lax.fori_loop(0, _ZR // 4, zrow, 0)

    for p in range(2):
        q = 2 * p + c
        col = l * H_DIM + p * 2 * QC

        if first:
            pltpu.sync_copy(hm_hbm.at[pl.ds(q * N_NODES, 1)], hrow)

        def start_loads(e0, b):
            if not first:
                pltpu.async_copy(src4_hbm.at[pl.ds(q * E_EDGES + e0, _K)],
                                 idx_s[b], sem_ld[b])
            pltpu.async_copy(dst_hbm.at[pl.ds(e0, _K)], idx_d[b], sem_ld[b])
            pltpu.async_copy(
                efp_hbm.at[pl.ds(e0, _K), pl.ds(col + c * QC, QC)],
                ebuf[b], sem_ld[b])

        def wait_loads(e0, b):
            if not first:
                pltpu.make_async_copy(src4_hbm.at[pl.ds(q * E_EDGES + e0, _K)],
                                      idx_s[b], sem_ld[b]).wait()
            pltpu.make_async_copy(dst_hbm.at[pl.ds(e0, _K)], idx_d[b],
                                  sem_ld[b]).wait()
            pltpu.make_async_copy(
                efp_hbm.at[pl.ds(e0, _K), pl.ds(col + c * QC, QC)],
                ebuf[b], sem_ld[b]).wait()

        def start_gather(b):
            if not first:
                pltpu.async_copy(hm_hbm.at[idx_s[b]], gbuf[b], sem_g[b])

        def wait_gather(b):
            if not first:
                pltpu.make_async_copy(hm_hbm.at[idx_s[b]], gbuf[b],
                                      sem_g[b]).wait()

        def start_scatter(b):
            pltpu.sync_copy(gbuf[b], aggsh.at[idx_d[b]], add=True)

        def compute(b):
            if first:
                row = hrow[0, 0:16]

                def frow0(r, cc):
                    for t in range(8):
                        x = row + ebuf[b][r * 8 + t, 0:16]
                        gbuf[b][r * 8 + t, 0:16] = x / (1.0 + jnp.exp(-x))
                    return cc

                lax.fori_loop(0, _K // 8, frow0, 0)
                return

            def frow(r, cc):
                for t in range(8):
                    x = gbuf[b][r * 8 + t, 0:16] + ebuf[b][r * 8 + t, 0:16]
                    gbuf[b][r * 8 + t, 0:16] = x / (1.0 + jnp.exp(-x))
                return cc

            lax.fori_loop(0, _K // 8, frow, 0)

        @pl.when(s < _NS - 1)
        def _():
            pltpu.sync_copy(zbuf, aggsh.at[pl.ds(s * _ZR, _ZR)])

        @pl.when(s == _NS - 1)
        def _():
            pltpu.sync_copy(zbuf.at[pl.ds(0, _ZR_LAST)],
                            aggsh.at[pl.ds((_NS - 1) * _ZR, _ZR_LAST)])

        plsc.subcore_barrier()

        start_loads(base, 0)
        start_loads(base + _K, 1)
        wait_loads(base, 0)
        start_gather(0)

        def step(j, u):
            v1 = (u + 1) % _NB
            v2 = (u + 2) % _NB
            wait_gather(u)

            @pl.when(j + 1 < _NCHUNK)
            def _():
                wait_loads(base + (j + 1) * _K, v1)
                start_gather(v1)

            compute(u)
            start_scatter(u)

            @pl.when(j + 2 < _NCHUNK)
            def _():
                start_loads(base + (j + 2) * _K, v2)

        def group(g, carry):
            for u in range(_NB):
                step(g * _NB + u, u)
            return carry

        lax.fori_loop(0, _NCHUNK // _NB, group, 0)

        @pl.when(s < _NEXTRA)
        def _():
            ex0 = (_NS * _NCHUNK + s) * _K
            start_loads(ex0, 0)
            wait_loads(ex0, 0)
            start_gather(0)
            wait_gather(0)
            compute(0)
            start_scatter(0)

        plsc.subcore_barrier()

        @pl.when(s == 0)
        def _():
            pltpu.sync_copy(aggsh, agg_hbm.at[q])

        plsc.subcore_barrier()


def _make_sc_edge(l):
    return pl.kernel(
        functools.partial(_sc_edge_body, l),
        out_type=jax.ShapeDtypeStruct((NQ, N_NODES, QC), jnp.float32),
        mesh=plsc.VectorSubcoreMesh(core_axis_name="c", subcore_axis_name="s"),
        compiler_params=pltpu.CompilerParams(use_tc_tiling_on_sc=False),
        scratch_types=(
            [pltpu.VMEM((_K,), jnp.int32) for _ in range(2 * _NB)]
            + [pltpu.VMEM((_K, QC), jnp.float32) for _ in range(2 * _NB)]
            + [pltpu.VMEM((_ZR, QC), jnp.float32)]
            + [pltpu.SemaphoreType.DMA for _ in range(3 * _NB)]
            + [pltpu.VMEM_SHARED((N_NODES, QC), jnp.float32)]
            + [pltpu.VMEM((1, QC), jnp.float32)]
        ),
    )


_SC_EDGE = [_make_sc_edge(l) for l in range(N_LAYERS)]


def _efp_body(ef_ref, w_ref, out_ref):
    out_ref[...] = jnp.dot(ef_ref[...], w_ref[...],
                           preferred_element_type=jnp.float32)


_efp_call = pl.pallas_call(
    _efp_body,
    grid=(E_EDGES // _TE,),
    in_specs=[pl.BlockSpec((_TE, ED_DIM), lambda i: (i, 0)),
              pl.BlockSpec((ED_DIM, N_LAYERS * H_DIM), lambda i: (0, 0))],
    out_specs=pl.BlockSpec((_TE, N_LAYERS * H_DIM), lambda i: (i, 0)),
    out_shape=jax.ShapeDtypeStruct((E_EDGES, N_LAYERS * H_DIM), jnp.float32),
)


def _init_body(win_ref, bin_ref, wma_ref, bm0_ref, h_ref, hm_ref):
    row = win_ref[...] + bin_ref[...]
    h_ref[...] = jnp.broadcast_to(row, (_TN, H_DIM))
    hmrow = jnp.dot(row, wma_ref[...], preferred_element_type=jnp.float32) + bm0_ref[...]
    for q in range(NQ):
        hm_ref[q] = jnp.broadcast_to(hmrow[:, q * QC:(q + 1) * QC], (_TN, QC))


_init_call = pl.pallas_call(
    _init_body,
    grid=(N_NODES // _TN,),
    in_specs=[pl.BlockSpec((1, H_DIM), lambda i: (0, 0)),
              pl.BlockSpec((1, H_DIM), lambda i: (0, 0)),
              pl.BlockSpec((H_DIM, H_DIM), lambda i: (0, 0)),
              pl.BlockSpec((1, H_DIM), lambda i: (0, 0))],
    out_specs=[pl.BlockSpec((_TN, H_DIM), lambda i: (i, 0)),
               pl.BlockSpec((NQ, _TN, QC), lambda i: (0, i, 0))],
    out_shape=[jax.ShapeDtypeStruct((N_NODES, H_DIM), jnp.float32),
               jax.ShapeDtypeStruct((NQ, N_NODES, QC), jnp.float32)],
)


def _update(h_ref, agg_ref, wu_ref, bu_ref):
    h = h_ref[...]
    u = jnp.dot(h, wu_ref[0:H_DIM], preferred_element_type=jnp.float32) + bu_ref[...]
    for q in range(NQ):
        u = u + jnp.dot(agg_ref[q], wu_ref[H_DIM + q * QC:H_DIM + (q + 1) * QC],
                        preferred_element_type=jnp.float32)
    return h + _silu(u)


def _mid_body(h_ref, agg_ref, wu_ref, bu_ref, wman_ref, bmn_ref, ho_ref, hmo_ref):
    hn = _update(h_ref, agg_ref, wu_ref, bu_ref)
    ho_ref[...] = hn
    hm = jnp.dot(hn, wman_ref[...], preferred_element_type=jnp.float32) + bmn_ref[...]
    for q in range(NQ):
        hmo_ref[q] = hm[:, q * QC:(q + 1) * QC]


_mid_call = pl.pallas_call(
    _mid_body,
    grid=(N_NODES // _TN,),
    in_specs=[pl.BlockSpec((_TN, H_DIM), lambda i: (i, 0)),
              pl.BlockSpec((NQ, _TN, QC), lambda i: (0, i, 0)),
              pl.BlockSpec((2 * H_DIM, H_DIM), lambda i: (0, 0)),
              pl.BlockSpec((1, H_DIM), lambda i: (0, 0)),
              pl.BlockSpec((H_DIM, H_DIM), lambda i: (0, 0)),
              pl.BlockSpec((1, H_DIM), lambda i: (0, 0))],
    out_specs=[pl.BlockSpec((_TN, H_DIM), lambda i: (i, 0)),
               pl.BlockSpec((NQ, _TN, QC), lambda i: (0, i, 0))],
    out_shape=[jax.ShapeDtypeStruct((N_NODES, H_DIM), jnp.float32),
               jax.ShapeDtypeStruct((NQ, N_NODES, QC), jnp.float32)],
)


def _final_body(h_ref, agg_ref, wu_ref, bu_ref, wr1_ref, br1_ref, wr2_ref,
                br2_ref, out_ref):
    hn = _update(h_ref, agg_ref, wu_ref, bu_ref)
    t = _silu(jnp.dot(hn, wr1_ref[...], preferred_element_type=jnp.float32)
              + br1_ref[...])
    out_ref[...] = jnp.dot(t, wr2_ref[...], preferred_element_type=jnp.float32) + br2_ref[...]


_final_call = pl.pallas_call(
    _final_body,
    grid=(N_NODES // _TN,),
    in_specs=[pl.BlockSpec((_TN, H_DIM), lambda i: (i, 0)),
              pl.BlockSpec((NQ, _TN, QC), lambda i: (0, i, 0)),
              pl.BlockSpec((2 * H_DIM, H_DIM), lambda i: (0, 0)),
              pl.BlockSpec((1, H_DIM), lambda i: (0, 0)),
              pl.BlockSpec((H_DIM, H_DIM), lambda i: (0, 0)),
              pl.BlockSpec((1, H_DIM), lambda i: (0, 0)),
              pl.BlockSpec((H_DIM, 1), lambda i: (0, 0)),
              pl.BlockSpec((1, 1), lambda i: (0, 0))],
    out_specs=pl.BlockSpec((_TN, 1), lambda i: (i, 0)),
    out_shape=jax.ShapeDtypeStruct((N_NODES, 1), jnp.float32),
)


def _softmax_body(x_ref, o_ref):
    x = x_ref[...]
    m = jnp.max(x)
    e = jnp.exp(x - m)
    o_ref[...] = e / jnp.sum(e)


_softmax_call = pl.pallas_call(
    _softmax_body,
    out_shape=jax.ShapeDtypeStruct((400, 125), jnp.float32),
)


def kernel(edge_index, edge_feat, N, W_in, b_in, Wm, bm, Wu, bu, Wr1, br1,
           Wr2, br2):
    del N
    src = edge_index[0].astype(jnp.int32)
    dst = edge_index[1].astype(jnp.int32)
    src4 = jnp.concatenate([src + q * N_NODES for q in range(NQ)])

    wmb = jnp.concatenate([Wm[l, H_DIM:, :] for l in range(N_LAYERS)], axis=1)
    efp = _efp_call(edge_feat, wmb)

    h, hm = _init_call(W_in, b_in.reshape(1, H_DIM), Wm[0, :H_DIM, :], bm[0:1])

    logits = None
    for l in range(N_LAYERS):
        agg = _SC_EDGE[l](hm.reshape(NQ * N_NODES, QC), efp, src4, dst)
        if l + 1 < N_LAYERS:
            h, hm = _mid_call(h, agg, Wu[l], bu[l:l + 1],
                              Wm[l + 1, :H_DIM, :], bm[l + 1:l + 2])
        else:
            logits = _final_call(h, agg, Wu[l], bu[l:l + 1], Wr1,
                                 br1.reshape(1, H_DIM), Wr2, br2.reshape(1, 1))

    p = _softmax_call(logits.reshape(400, 125))
    return p.reshape(N_NODES)


# speedup vs baseline: 1.8448x; 1.3288x over previous
import functools

import jax
import jax.numpy as jnp
from jax import lax
from jax.experimental import pallas as pl
from jax.experimental.pallas import tpu as pltpu
from jax.experimental.pallas import tpu_sc as plsc

N_NODES = 50000
E_EDGES = 800000
H_DIM = 64
ED_DIM = 16
N_LAYERS = 6
QC = 16
NQ = 4

_NC = 2
_NS = 16
_K = 128
_NCG = E_EDGES // _K
_NCHUNK = _NCG // _NS
_NEXTRA = _NCG - _NCHUNK * _NS
_ZR = 3128
_ZR_LAST = N_NODES - (_NS - 1) * _ZR

_TE = 4000
_TN = 2000


def _silu(x):
    return x / (1.0 + jnp.exp(-x))


_NB = 5


def _sc_edge_body(l, hm_hbm, efp_hbm, src4_hbm, dst_hbm, agg_hbm, *scr):
    first = l == 0
    idx_s = scr[0:_NB]
    idx_d = scr[_NB:2 * _NB]
    gbuf = scr[2 * _NB:3 * _NB]
    ebuf = scr[3 * _NB:4 * _NB]
    zbuf = scr[4 * _NB]
    sem_ld = scr[4 * _NB + 1:5 * _NB + 1]
    sem_g = scr[5 * _NB + 1:6 * _NB + 1]
    sem_sc = scr[6 * _NB + 1:7 * _NB + 1]
    aggsh = scr[7 * _NB + 1]
    hrow = scr[7 * _NB + 2]

    c = lax.axis_index("c")
    s = lax.axis_index("s")
    base = s * _NCHUNK * _K

    def zrow(j, carry):
        for r in range(4):
            zbuf[j * 4 + r, 0:16] = jnp.zeros((16,), jnp.float32)
        return carry

    lax.fori_loop(0, _ZR // 4, zrow, 0)

    for p in range(2):
        q = 2 * p + c
        col = l * H_DIM + p * 2 * QC

        if first:
            pltpu.sync_copy(hm_hbm.at[pl.ds(q * N_NODES, 1)], hrow)

        def start_loads(e0, b):
            if not first:
                pltpu.async_copy(src4_hbm.at[pl.ds(q * E_EDGES + e0, _K)],
                                 idx_s[b], sem_ld[b])
            pltpu.async_copy(dst_hbm.at[pl.ds(e0, _K)], idx_d[b], sem_ld[b])
            pltpu.async_copy(
                efp_hbm.at[pl.ds(e0, _K), pl.ds(col + c * QC, QC)],
                ebuf[b], sem_ld[b])

        def wait_loads(e0, b):
            if not first:
                pltpu.make_async_copy(src4_hbm.at[pl.ds(q * E_EDGES + e0, _K)],
                                      idx_s[b], sem_ld[b]).wait()
            pltpu.make_async_copy(dst_hbm.at[pl.ds(e0, _K)], idx_d[b],
                                  sem_ld[b]).wait()
            pltpu.make_async_copy(
                efp_hbm.at[pl.ds(e0, _K), pl.ds(col + c * QC, QC)],
                ebuf[b], sem_ld[b]).wait()

        def start_gather(b):
            if not first:
                pltpu.async_copy(hm_hbm.at[idx_s[b]], gbuf[b], sem_g[b])

        def wait_gather(b):
            if not first:
                pltpu.make_async_copy(hm_hbm.at[idx_s[b]], gbuf[b],
                                      sem_g[b]).wait()

        def start_scatter(b):
            pltpu.sync_copy(gbuf[b], aggsh.at[idx_d[b]], add=True)

        def compute(b):
            if first:
                row = hrow[0, 0:16]

                def frow0(r, cc):
                    for t in range(8):
                        x = row + ebuf[b][r * 8 + t, 0:16]
                        gbuf[b][r * 8 + t, 0:16] = x / (1.0 + jnp.exp(-x))
                    return cc

                lax.fori_loop(0, _K // 8, frow0, 0)
                return

            def frow(r, cc):
                for t in range(8):
                    x = gbuf[b][r * 8 + t, 0:16] + ebuf[b][r * 8 + t, 0:16]
                    gbuf[b][r * 8 + t, 0:16] = x / (1.0 + jnp.exp(-x))
                return cc

            lax.fori_loop(0, _K // 8, frow, 0)

        @pl.when(s < _NS - 1)
        def _():
            pltpu.sync_copy(zbuf, aggsh.at[pl.ds(s * _ZR, _ZR)])

        @pl.when(s == _NS - 1)
        def _():
            pltpu.sync_copy(zbuf.at[pl.ds(0, _ZR_LAST)],
                            aggsh.at[pl.ds((_NS - 1) * _ZR, _ZR_LAST)])

        plsc.subcore_barrier()

        start_loads(base, 0)
        start_loads(base + _K, 1)
        start_loads(base + 2 * _K, 2)
        wait_loads(base, 0)
        start_gather(0)

        def step(j, u):
            v1 = (u + 1) % _NB
            v3 = (u + 3) % _NB
            wait_gather(u)

            @pl.when(j + 1 < _NCHUNK)
            def _():
                wait_loads(base + (j + 1) * _K, v1)
                start_gather(v1)

            compute(u)
            start_scatter(u)

            @pl.when(j + 3 < _NCHUNK)
            def _():
                start_loads(base + (j + 3) * _K, v3)

        def group(g, carry):
            for u in range(_NB):
                step(g * _NB + u, u)
            return carry

        lax.fori_loop(0, _NCHUNK // _NB, group, 0)

        @pl.when(s < _NEXTRA)
        def _():
            ex0 = (_NS * _NCHUNK + s) * _K
            start_loads(ex0, 0)
            wait_loads(ex0, 0)
            start_gather(0)
            wait_gather(0)
            compute(0)
            start_scatter(0)

        plsc.subcore_barrier()

        @pl.when(s == 0)
        def _():
            pltpu.sync_copy(aggsh, agg_hbm.at[q])

        plsc.subcore_barrier()


def _make_sc_edge(l):
    return pl.kernel(
        functools.partial(_sc_edge_body, l),
        out_type=jax.ShapeDtypeStruct((NQ, N_NODES, QC), jnp.float32),
        mesh=plsc.VectorSubcoreMesh(core_axis_name="c", subcore_axis_name="s"),
        compiler_params=pltpu.CompilerParams(use_tc_tiling_on_sc=False),
        scratch_types=(
            [pltpu.VMEM((_K,), jnp.int32) for _ in range(2 * _NB)]
            + [pltpu.VMEM((_K, QC), jnp.float32) for _ in range(2 * _NB)]
            + [pltpu.VMEM((_ZR, QC), jnp.float32)]
            + [pltpu.SemaphoreType.DMA for _ in range(3 * _NB)]
            + [pltpu.VMEM_SHARED((N_NODES, QC), jnp.float32)]
            + [pltpu.VMEM((1, QC), jnp.float32)]
        ),
    )


_SC_EDGE = [_make_sc_edge(l) for l in range(N_LAYERS)]


def _efp_body(ef_ref, w_ref, out_ref):
    out_ref[...] = jnp.dot(ef_ref[...], w_ref[...],
                           preferred_element_type=jnp.float32)


_efp_call = pl.pallas_call(
    _efp_body,
    grid=(E_EDGES // _TE,),
    in_specs=[pl.BlockSpec((_TE, ED_DIM), lambda i: (i, 0)),
              pl.BlockSpec((ED_DIM, N_LAYERS * H_DIM), lambda i: (0, 0))],
    out_specs=pl.BlockSpec((_TE, N_LAYERS * H_DIM), lambda i: (i, 0)),
    out_shape=jax.ShapeDtypeStruct((E_EDGES, N_LAYERS * H_DIM), jnp.float32),
)


def _init_body(win_ref, bin_ref, wma_ref, bm0_ref, h_ref, hm_ref):
    row = win_ref[...] + bin_ref[...]
    h_ref[...] = jnp.broadcast_to(row, (_TN, H_DIM))
    hmrow = jnp.dot(row, wma_ref[...], preferred_element_type=jnp.float32) + bm0_ref[...]
    for q in range(NQ):
        hm_ref[q] = jnp.broadcast_to(hmrow[:, q * QC:(q + 1) * QC], (_TN, QC))


_init_call = pl.pallas_call(
    _init_body,
    grid=(N_NODES // _TN,),
    in_specs=[pl.BlockSpec((1, H_DIM), lambda i: (0, 0)),
              pl.BlockSpec((1, H_DIM), lambda i: (0, 0)),
              pl.BlockSpec((H_DIM, H_DIM), lambda i: (0, 0)),
              pl.BlockSpec((1, H_DIM), lambda i: (0, 0))],
    out_specs=[pl.BlockSpec((_TN, H_DIM), lambda i: (i, 0)),
               pl.BlockSpec((NQ, _TN, QC), lambda i: (0, i, 0))],
    out_shape=[jax.ShapeDtypeStruct((N_NODES, H_DIM), jnp.float32),
               jax.ShapeDtypeStruct((NQ, N_NODES, QC), jnp.float32)],
)


def _update(h_ref, agg_ref, wu_ref, bu_ref):
    h = h_ref[...]
    u = jnp.dot(h, wu_ref[0:H_DIM], preferred_element_type=jnp.float32) + bu_ref[...]
    for q in range(NQ):
        u = u + jnp.dot(agg_ref[q], wu_ref[H_DIM + q * QC:H_DIM + (q + 1) * QC],
                        preferred_element_type=jnp.float32)
    return h + _silu(u)


def _mid_body(h_ref, agg_ref, wu_ref, bu_ref, wman_ref, bmn_ref, ho_ref, hmo_ref):
    hn = _update(h_ref, agg_ref, wu_ref, bu_ref)
    ho_ref[...] = hn
    hm = jnp.dot(hn, wman_ref[...], preferred_element_type=jnp.float32) + bmn_ref[...]
    for q in range(NQ):
        hmo_ref[q] = hm[:, q * QC:(q + 1) * QC]


_mid_call = pl.pallas_call(
    _mid_body,
    grid=(N_NODES // _TN,),
    in_specs=[pl.BlockSpec((_TN, H_DIM), lambda i: (i, 0)),
              pl.BlockSpec((NQ, _TN, QC), lambda i: (0, i, 0)),
              pl.BlockSpec((2 * H_DIM, H_DIM), lambda i: (0, 0)),
              pl.BlockSpec((1, H_DIM), lambda i: (0, 0)),
              pl.BlockSpec((H_DIM, H_DIM), lambda i: (0, 0)),
              pl.BlockSpec((1, H_DIM), lambda i: (0, 0))],
    out_specs=[pl.BlockSpec((_TN, H_DIM), lambda i: (i, 0)),
               pl.BlockSpec((NQ, _TN, QC), lambda i: (0, i, 0))],
    out_shape=[jax.ShapeDtypeStruct((N_NODES, H_DIM), jnp.float32),
               jax.ShapeDtypeStruct((NQ, N_NODES, QC), jnp.float32)],
)


def _final_body(h_ref, agg_ref, wu_ref, bu_ref, wr1_ref, br1_ref, wr2_ref,
                br2_ref, out_ref):
    hn = _update(h_ref, agg_ref, wu_ref, bu_ref)
    t = _silu(jnp.dot(hn, wr1_ref[...], preferred_element_type=jnp.float32)
              + br1_ref[...])
    out_ref[...] = jnp.dot(t, wr2_ref[...], preferred_element_type=jnp.float32) + br2_ref[...]


_final_call = pl.pallas_call(
    _final_body,
    grid=(N_NODES // _TN,),
    in_specs=[pl.BlockSpec((_TN, H_DIM), lambda i: (i, 0)),
              pl.BlockSpec((NQ, _TN, QC), lambda i: (0, i, 0)),
              pl.BlockSpec((2 * H_DIM, H_DIM), lambda i: (0, 0)),
              pl.BlockSpec((1, H_DIM), lambda i: (0, 0)),
              pl.BlockSpec((H_DIM, H_DIM), lambda i: (0, 0)),
              pl.BlockSpec((1, H_DIM), lambda i: (0, 0)),
              pl.BlockSpec((H_DIM, 1), lambda i: (0, 0)),
              pl.BlockSpec((1, 1), lambda i: (0, 0))],
    out_specs=pl.BlockSpec((_TN, 1), lambda i: (i, 0)),
    out_shape=jax.ShapeDtypeStruct((N_NODES, 1), jnp.float32),
)


def _softmax_body(x_ref, o_ref):
    x = x_ref[...]
    m = jnp.max(x)
    e = jnp.exp(x - m)
    o_ref[...] = e / jnp.sum(e)


_softmax_call = pl.pallas_call(
    _softmax_body,
    out_shape=jax.ShapeDtypeStruct((400, 125), jnp.float32),
)


def kernel(edge_index, edge_feat, N, W_in, b_in, Wm, bm, Wu, bu, Wr1, br1,
           Wr2, br2):
    del N
    src = edge_index[0].astype(jnp.int32)
    dst = edge_index[1].astype(jnp.int32)
    src4 = jnp.concatenate([src + q * N_NODES for q in range(NQ)])

    wmb = jnp.concatenate([Wm[l, H_DIM:, :] for l in range(N_LAYERS)], axis=1)
    efp = _efp_call(edge_feat, wmb)

    h, hm = _init_call(W_in, b_in.reshape(1, H_DIM), Wm[0, :H_DIM, :], bm[0:1])

    logits = None
    for l in range(N_LAYERS):
        agg = _SC_EDGE[l](hm.reshape(NQ * N_NODES, QC), efp, src4, dst)
        if l + 1 < N_LAYERS:
            h, hm = _mid_call(h, agg, Wu[l], bu[l:l + 1],
                              Wm[l + 1, :H_DIM, :], bm[l + 1:l + 2])
        else:
            logits = _final_call(h, agg, Wu[l], bu[l:l + 1], Wr1,
                                 br1.reshape(1, H_DIM), Wr2, br2.reshape(1, 1))

    p = _softmax_call(logits.reshape(400, 125))
    return p.reshape(N_NODES)
